# speedup vs baseline: 1.1308x; 1.1308x over previous
import jax
import jax.numpy as jnp
from jax.experimental import pallas as pl
from jax.experimental.pallas import tpu as pltpu


def _fused_body(x_ref, w1t_ref, b1_ref, w2t_ref, b2_ref, o_ref):
    x = x_ref[0]
    inv_s = jnp.float32(1.0 / x.shape[1])
    pooled = jnp.sum(x, axis=1, keepdims=True,
                     dtype=jnp.float32) * inv_s
    h = jnp.maximum(
        jnp.dot(w1t_ref[...], pooled,
                preferred_element_type=jnp.float32) + b1_ref[...], 0.0)
    gate = jax.nn.sigmoid(
        jnp.dot(w2t_ref[...], h,
                preferred_element_type=jnp.float32) + b2_ref[...])
    o_ref[0] = (x * gate.astype(x.dtype)).astype(o_ref.dtype)


def kernel(x, w1, b1, w2, b2):
    N, C, D, H, W = x.shape
    S = D * H * W
    Ch = w1.shape[1]

    x_flat = x.reshape(N, C, S)
    w1t = w1.T.astype(jnp.float32)
    w2t = w2.T.astype(jnp.float32)
    b1c = b1.reshape(-1, 1).astype(jnp.float32)
    b2c = b2.reshape(-1, 1).astype(jnp.float32)

    out = pl.pallas_call(
        _fused_body,
        out_shape=jax.ShapeDtypeStruct((N, C, S), x.dtype),
        grid=(N,),
        in_specs=[
            pl.BlockSpec((1, C, S), lambda n: (n, 0, 0)),
            pl.BlockSpec((Ch, C), lambda n: (0, 0)),
            pl.BlockSpec((Ch, 1), lambda n: (0, 0)),
            pl.BlockSpec((C, Ch), lambda n: (0, 0)),
            pl.BlockSpec((C, 1), lambda n: (0, 0)),
        ],
        out_specs=pl.BlockSpec((1, C, S), lambda n: (n, 0, 0)),
        compiler_params=pltpu.CompilerParams(
            dimension_semantics=("parallel",),
            vmem_limit_bytes=48 * 1024 * 1024),
    )(x_flat, w1t, b1c, w2t, b2c)

    return out.reshape(N, C, D, H, W)


# speedup vs baseline: 4.0790x; 3.6071x over previous
import jax
import jax.numpy as jnp
from jax.experimental import pallas as pl
from jax.experimental.pallas import tpu as pltpu


def _fused_body(x_ref, w1_ref, b1_ref, w2_ref, b2_ref, o_ref):
    x = x_ref[0]
    inv_s = jnp.float32(1.0 / x.shape[0])
    pooled = jnp.sum(x, axis=0, keepdims=True,
                     dtype=jnp.float32) * inv_s
    h = jnp.maximum(
        jnp.dot(pooled, w1_ref[...],
                preferred_element_type=jnp.float32) + b1_ref[...], 0.0)
    gate = jax.nn.sigmoid(
        jnp.dot(h, w2_ref[...],
                preferred_element_type=jnp.float32) + b2_ref[...])
    o_ref[0] = (x * gate.astype(x.dtype)).astype(o_ref.dtype)


def kernel(x, w1, b1, w2, b2):
    N, C, D, H, W = x.shape
    S = D * H * W
    Ch = w1.shape[1]

    x_nsc = jnp.transpose(x, (0, 2, 3, 4, 1)).reshape(N, S, C)
    b1r = b1.reshape(1, Ch).astype(jnp.float32)
    b2r = b2.reshape(1, C).astype(jnp.float32)

    out = pl.pallas_call(
        _fused_body,
        out_shape=jax.ShapeDtypeStruct((N, S, C), x.dtype),
        grid=(N,),
        in_specs=[
            pl.BlockSpec((1, S, C), lambda n: (n, 0, 0)),
            pl.BlockSpec((C, Ch), lambda n: (0, 0)),
            pl.BlockSpec((1, Ch), lambda n: (0, 0)),
            pl.BlockSpec((Ch, C), lambda n: (0, 0)),
            pl.BlockSpec((1, C), lambda n: (0, 0)),
        ],
        out_specs=pl.BlockSpec((1, S, C), lambda n: (n, 0, 0)),
        compiler_params=pltpu.CompilerParams(
            dimension_semantics=("parallel",),
            vmem_limit_bytes=48 * 1024 * 1024),
    )(x_nsc, w1.astype(jnp.float32), b1r, w2.astype(jnp.float32), b2r)

    return out.reshape(N, D, H, W, C).transpose(0, 4, 1, 2, 3)
